# baseline (speedup 1.0000x reference)
import functools

import jax
import jax.numpy as jnp
from jax import lax
from jax.experimental import pallas as pl
from jax.experimental.pallas import tpu as pltpu
from jax.experimental.pallas import tpu_sc as plsc

N = 10000
E = 320000
NSTEP = 3
NFEAT = 128
NHID = 128
NCLASS = 64

NC = 2
NS = 16
LANES = 16

NP = 10240
RPT = NP // NS
ZR = 32
ET0 = E // (NC * NS)
C = 128
ET = 10240
NCH = ET // C

RB = 1000


def _support_body(x_ref, w_ref, o_ref):
    o_ref[...] = jnp.dot(x_ref[...], w_ref[0],
                         preferred_element_type=jnp.float32)


def _supports(x, W1):
    return pl.pallas_call(
        _support_body,
        grid=(NSTEP, N // RB),
        in_specs=[
            pl.BlockSpec((RB, NFEAT), lambda s, r: (r, 0)),
            pl.BlockSpec((1, NFEAT, NHID), lambda s, r: (s, 0, 0)),
        ],
        out_specs=pl.BlockSpec(
            (RB, NHID), lambda s, r: (s * (N // RB) + r, 0)),
        out_shape=jax.ShapeDtypeStruct((NSTEP * N, NHID), jnp.float32),
    )(x, W1)


NBUF = 2
EB = ET // 2
CHB = EB // C
NGB = CHB // NBUF


def _sc_spmm(nsteps, sup, srcp, dstp):
    mesh = plsc.VectorSubcoreMesh(core_axis_name="c", subcore_axis_name="s")

    @functools.partial(
        pl.kernel,
        mesh=mesh,
        out_type=jax.ShapeDtypeStruct((nsteps * NC * NP, NHID), jnp.float32),
        scratch_types=[
            pltpu.VMEM((EB,), jnp.int32),
            pltpu.VMEM((EB,), jnp.int32),
            pltpu.VMEM((NBUF, C, NHID), jnp.float32),
            pltpu.VMEM((ZR, NHID), jnp.float32),
            pltpu.VMEM_SHARED((NP, NHID), jnp.float32),
            pltpu.SemaphoreType.DMA,
            pltpu.SemaphoreType.DMA,
            pltpu.SemaphoreType.DMA,
            pltpu.SemaphoreType.DMA,
        ],
    )
    def k(sup_hbm, src_hbm, dst_hbm, out_hbm, src_v, dst_v, rows_v, zbuf,
          acc, g0, g1, s0, s1):
        c = lax.axis_index("c")
        s = lax.axis_index("s")
        gsem = [g0, g1]
        ssem = [s0, s1]

        def issue_gather(g, b):
            pltpu.async_copy(sup_hbm.at[src_v.at[pl.ds(g * C, C)]],
                             rows_v.at[b], gsem[b])

        def wait_gather(b):
            pltpu.make_async_copy(sup_hbm.at[pl.ds(0, C)], rows_v.at[b],
                                  gsem[b]).wait()

        def issue_scatter(g, b):
            pltpu.async_copy(rows_v.at[b], acc.at[dst_v.at[pl.ds(g * C, C)]],
                             ssem[b], add=True)

        def wait_scatter(b):
            pltpu.make_async_copy(rows_v.at[b], acc.at[pl.ds(0, C)],
                                  ssem[b]).wait()

        @pl.loop(0, ZR)
        def _(i):
            for j in range(NHID // LANES):
                zbuf[i, pl.ds(j * LANES, LANES)] = jnp.zeros(
                    (LANES,), jnp.float32)

        for st in range(nsteps):
            for kk in range(RPT // ZR):
                pltpu.sync_copy(zbuf, acc.at[pl.ds(s * RPT + kk * ZR, ZR)])
            plsc.subcore_barrier()

            base = ((st * NC + c) * NS + s) * ET
            for blk in range(ET // EB):
                boff = base + blk * EB
                pltpu.sync_copy(src_hbm.at[pl.ds(boff, EB)], src_v)
                pltpu.sync_copy(dst_hbm.at[pl.ds(boff, EB)], dst_v)

                issue_gather(0, 0)

                wait_gather(0)
                issue_scatter(0, 0)
                issue_gather(1, 1)
                wait_gather(1)
                issue_scatter(1, 1)
                wait_scatter(0)
                issue_gather(2, 0)

                @pl.loop(1, NGB - 1)
                def _(gg):
                    for b in range(NBUF):
                        g = gg * NBUF + b
                        wait_gather(b)
                        issue_scatter(g, b)
                        bh = (b + 1) % NBUF
                        wait_scatter(bh)
                        issue_gather(g + 1, bh)

                g = (NGB - 1) * NBUF
                wait_gather(0)
                issue_scatter(g, 0)
                wait_scatter(1)
                issue_gather(g + 1, 1)
                wait_gather(1)
                issue_scatter(g + 1, 1)

                for b in range(NBUF):
                    wait_scatter(b)

            plsc.subcore_barrier()
            pltpu.sync_copy(acc.at[pl.ds(s * RPT, RPT)],
                            out_hbm.at[pl.ds((st * NC + c) * NP + s * RPT,
                                             RPT)])

    return k(sup, srcp, dstp)


def _mid_body(p_ref, b1_ref, w_ref, o_ref):
    o = jnp.zeros((RB, NHID), jnp.float32)
    for s in range(NSTEP):
        h = jnp.maximum(p_ref[s, 0] + p_ref[s, 1] + b1_ref[s][None, :], 0.0)
        o = o + jnp.dot(h, w_ref[s], preferred_element_type=jnp.float32)
    o_ref[...] = o


def _mid(agg, b1, W2p):
    return pl.pallas_call(
        _mid_body,
        grid=(N // RB,),
        in_specs=[
            pl.BlockSpec((NSTEP, NC, RB, NHID), lambda r: (0, 0, r, 0)),
            pl.BlockSpec((NSTEP, NHID), lambda r: (0, 0)),
            pl.BlockSpec((NSTEP, NHID, NHID), lambda r: (0, 0, 0)),
        ],
        out_specs=pl.BlockSpec((RB, NHID), lambda r: (r, 0)),
        out_shape=jax.ShapeDtypeStruct((N, NHID), jnp.float32),
    )(agg, b1, W2p)


def _final_body(p_ref, b2_ref, o_ref):
    a = p_ref[0, :, :NCLASS] + p_ref[1, :, :NCLASS] + b2_ref[...]
    m = jnp.max(a, axis=1, keepdims=True)
    ex = jnp.exp(a - m)
    lse = jnp.log(jnp.sum(ex, axis=1, keepdims=True))
    o_ref[...] = a - m - lse


def _final(parts, b2):
    return pl.pallas_call(
        _final_body,
        grid=(N // RB,),
        in_specs=[
            pl.BlockSpec((NC, RB, NHID), lambda r: (0, r, 0)),
            pl.BlockSpec((1, NCLASS), lambda r: (0, 0)),
        ],
        out_specs=pl.BlockSpec((RB, NCLASS), lambda r: (r, 0)),
        out_shape=jax.ShapeDtypeStruct((N, NCLASS), jnp.float32),
    )(parts, b2)


def _pad_edges(src, dst):
    srcp = jnp.pad(src, ((0, 0), (0, 0), (0, ET - ET0)))
    dstp = jnp.pad(dst, ((0, 0), (0, 0), (0, ET - ET0)), constant_values=N)
    return srcp.reshape(-1), dstp.reshape(-1)


def kernel(x, adjs, W1, b1, W2, b2):
    sup1 = _supports(x, W1)

    step_off = (jnp.arange(NSTEP, dtype=jnp.int32) * N)[:, None]
    src1 = (adjs[:, 0, :] + step_off).reshape(NSTEP, NC * NS, ET0)
    dst1 = adjs[:, 1, :].reshape(NSTEP, NC * NS, ET0)
    src1, dst1 = _pad_edges(src1, dst1)
    agg = _sc_spmm(NSTEP, sup1, src1, dst1).reshape(NSTEP, NC, NP, NHID)

    W2p = jnp.pad(W2.reshape(NSTEP, NHID, NCLASS),
                  ((0, 0), (0, 0), (0, NHID - NCLASS)))
    sup2 = _mid(agg, b1, W2p)

    src2 = adjs[0, 0].reshape(1, NC * NS, ET0)
    dst2 = adjs[0, 1].reshape(1, NC * NS, ET0)
    src2, dst2 = _pad_edges(src2, dst2)
    parts = _sc_spmm(1, sup2, src2, dst2).reshape(NC, NP, NHID)

    return _final(parts, b2.reshape(1, NCLASS))


# speedup vs baseline: 1.0332x; 1.0332x over previous
import functools

import jax
import jax.numpy as jnp
from jax import lax
from jax.experimental import pallas as pl
from jax.experimental.pallas import tpu as pltpu
from jax.experimental.pallas import tpu_sc as plsc

N = 10000
E = 320000
NSTEP = 3
NFEAT = 128
NHID = 128
NCLASS = 64

NC = 2
NS = 16
LANES = 16

NP = 10240
RPT = NP // NS
ZR = 32
ET0 = E // (NC * NS)
C = 128
ET = 10240
NCH = ET // C

RB = 1000


def _support_body(x_ref, w_ref, o_ref):
    o_ref[...] = jnp.dot(x_ref[...], w_ref[0],
                         preferred_element_type=jnp.float32)


def _supports(x, W1):
    return pl.pallas_call(
        _support_body,
        grid=(NSTEP, N // RB),
        in_specs=[
            pl.BlockSpec((RB, NFEAT), lambda s, r: (r, 0)),
            pl.BlockSpec((1, NFEAT, NHID), lambda s, r: (s, 0, 0)),
        ],
        out_specs=pl.BlockSpec(
            (RB, NHID), lambda s, r: (s * (N // RB) + r, 0)),
        out_shape=jax.ShapeDtypeStruct((NSTEP * N, NHID), jnp.float32),
    )(x, W1)


CC = 32
NBUF = 8
LKA = 4
EB = ET // 2
CHB = EB // CC
NGB = CHB // NBUF


def _sc_spmm(nsteps, sup, srcp, dstp):
    mesh = plsc.VectorSubcoreMesh(core_axis_name="c", subcore_axis_name="s")

    @functools.partial(
        pl.kernel,
        mesh=mesh,
        out_type=jax.ShapeDtypeStruct((nsteps * NC * NP, NHID), jnp.float32),
        scratch_types=[
            pltpu.VMEM((EB,), jnp.int32),
            pltpu.VMEM((EB,), jnp.int32),
            pltpu.VMEM((NBUF, CC, NHID), jnp.float32),
            pltpu.VMEM((ZR, NHID), jnp.float32),
            pltpu.VMEM_SHARED((NP, NHID), jnp.float32),
        ] + [pltpu.SemaphoreType.DMA] * (2 * NBUF),
    )
    def k(sup_hbm, src_hbm, dst_hbm, out_hbm, src_v, dst_v, rows_v, zbuf,
          acc, *sems):
        c = lax.axis_index("c")
        s = lax.axis_index("s")
        gsem = sems[:NBUF]
        ssem = sems[NBUF:]

        def issue_gather(g, b):
            pltpu.async_copy(sup_hbm.at[src_v.at[pl.ds(g * CC, CC)]],
                             rows_v.at[b], gsem[b])

        def wait_gather(b):
            pltpu.make_async_copy(sup_hbm.at[pl.ds(0, CC)], rows_v.at[b],
                                  gsem[b]).wait()

        def issue_scatter(g, b):
            pltpu.async_copy(rows_v.at[b], acc.at[dst_v.at[pl.ds(g * CC, CC)]],
                             ssem[b], add=True)

        def wait_scatter(b):
            pltpu.make_async_copy(rows_v.at[b], acc.at[pl.ds(0, CC)],
                                  ssem[b]).wait()

        @pl.loop(0, ZR)
        def _(i):
            for j in range(NHID // LANES):
                zbuf[i, pl.ds(j * LANES, LANES)] = jnp.zeros(
                    (LANES,), jnp.float32)

        for st in range(nsteps):
            for kk in range(RPT // ZR):
                pltpu.sync_copy(zbuf, acc.at[pl.ds(s * RPT + kk * ZR, ZR)])
            plsc.subcore_barrier()

            base = ((st * NC + c) * NS + s) * ET
            for blk in range(ET // EB):
                boff = base + blk * EB
                pltpu.sync_copy(src_hbm.at[pl.ds(boff, EB)], src_v)
                pltpu.sync_copy(dst_hbm.at[pl.ds(boff, EB)], dst_v)

                for g in range(LKA):
                    issue_gather(g, g)

                for b in range(NBUF):
                    g, h = b, b + LKA
                    wait_gather(b)
                    issue_scatter(g, b)
                    bh = h % NBUF
                    if h >= NBUF:
                        wait_scatter(bh)
                    issue_gather(h, bh)

                @pl.loop(1, NGB - 1)
                def _(gg):
                    for b in range(NBUF):
                        g = gg * NBUF + b
                        wait_gather(b)
                        issue_scatter(g, b)
                        bh = (b + LKA) % NBUF
                        wait_scatter(bh)
                        issue_gather(g + LKA, bh)

                for b in range(NBUF):
                    g = (NGB - 1) * NBUF + b
                    wait_gather(b)
                    issue_scatter(g, b)
                    if g + LKA < CHB:
                        bh = (b + LKA) % NBUF
                        wait_scatter(bh)
                        issue_gather(g + LKA, bh)

                for b in range(NBUF):
                    wait_scatter(b)

            plsc.subcore_barrier()
            pltpu.sync_copy(acc.at[pl.ds(s * RPT, RPT)],
                            out_hbm.at[pl.ds((st * NC + c) * NP + s * RPT,
                                             RPT)])

    return k(sup, srcp, dstp)


def _mid_body(p_ref, b1_ref, w_ref, o_ref):
    o = jnp.zeros((RB, NHID), jnp.float32)
    for s in range(NSTEP):
        h = jnp.maximum(p_ref[s, 0] + p_ref[s, 1] + b1_ref[s][None, :], 0.0)
        o = o + jnp.dot(h, w_ref[s], preferred_element_type=jnp.float32)
    o_ref[...] = o


def _mid(agg, b1, W2p):
    return pl.pallas_call(
        _mid_body,
        grid=(N // RB,),
        in_specs=[
            pl.BlockSpec((NSTEP, NC, RB, NHID), lambda r: (0, 0, r, 0)),
            pl.BlockSpec((NSTEP, NHID), lambda r: (0, 0)),
            pl.BlockSpec((NSTEP, NHID, NHID), lambda r: (0, 0, 0)),
        ],
        out_specs=pl.BlockSpec((RB, NHID), lambda r: (r, 0)),
        out_shape=jax.ShapeDtypeStruct((N, NHID), jnp.float32),
    )(agg, b1, W2p)


def _final_body(p_ref, b2_ref, o_ref):
    a = p_ref[0, :, :NCLASS] + p_ref[1, :, :NCLASS] + b2_ref[...]
    m = jnp.max(a, axis=1, keepdims=True)
    ex = jnp.exp(a - m)
    lse = jnp.log(jnp.sum(ex, axis=1, keepdims=True))
    o_ref[...] = a - m - lse


def _final(parts, b2):
    return pl.pallas_call(
        _final_body,
        grid=(N // RB,),
        in_specs=[
            pl.BlockSpec((NC, RB, NHID), lambda r: (0, r, 0)),
            pl.BlockSpec((1, NCLASS), lambda r: (0, 0)),
        ],
        out_specs=pl.BlockSpec((RB, NCLASS), lambda r: (r, 0)),
        out_shape=jax.ShapeDtypeStruct((N, NCLASS), jnp.float32),
    )(parts, b2)


def _pad_edges(src, dst):
    srcp = jnp.pad(src, ((0, 0), (0, 0), (0, ET - ET0)))
    dstp = jnp.pad(dst, ((0, 0), (0, 0), (0, ET - ET0)), constant_values=N)
    return srcp.reshape(-1), dstp.reshape(-1)


def kernel(x, adjs, W1, b1, W2, b2):
    sup1 = _supports(x, W1)

    step_off = (jnp.arange(NSTEP, dtype=jnp.int32) * N)[:, None]
    src1 = (adjs[:, 0, :] + step_off).reshape(NSTEP, NC * NS, ET0)
    dst1 = adjs[:, 1, :].reshape(NSTEP, NC * NS, ET0)
    src1, dst1 = _pad_edges(src1, dst1)
    agg = _sc_spmm(NSTEP, sup1, src1, dst1).reshape(NSTEP, NC, NP, NHID)

    W2p = jnp.pad(W2.reshape(NSTEP, NHID, NCLASS),
                  ((0, 0), (0, 0), (0, NHID - NCLASS)))
    sup2 = _mid(agg, b1, W2p)

    src2 = adjs[0, 0].reshape(1, NC * NS, ET0)
    dst2 = adjs[0, 1].reshape(1, NC * NS, ET0)
    src2, dst2 = _pad_edges(src2, dst2)
    parts = _sc_spmm(1, sup2, src2, dst2).reshape(NC, NP, NHID)

    return _final(parts, b2.reshape(1, NCLASS))


# speedup vs baseline: 1.0484x; 1.0148x over previous
import functools

import jax
import jax.numpy as jnp
from jax import lax
from jax.experimental import pallas as pl
from jax.experimental.pallas import tpu as pltpu
from jax.experimental.pallas import tpu_sc as plsc

N = 10000
E = 320000
NSTEP = 3
NFEAT = 128
NHID = 128
NCLASS = 64

NC = 2
NS = 16
LANES = 16

NP = 10240
RPT = NP // NS
ZR = 32
ET0 = E // (NC * NS)
C = 128
ET = 10240
NCH = ET // C

RB = 1000


def _support_body(x_ref, w_ref, o_ref):
    o_ref[...] = jnp.dot(x_ref[...], w_ref[0],
                         preferred_element_type=jnp.float32)


def _supports(x, W1):
    return pl.pallas_call(
        _support_body,
        grid=(NSTEP, N // RB),
        in_specs=[
            pl.BlockSpec((RB, NFEAT), lambda s, r: (r, 0)),
            pl.BlockSpec((1, NFEAT, NHID), lambda s, r: (s, 0, 0)),
        ],
        out_specs=pl.BlockSpec(
            (RB, NHID), lambda s, r: (s * (N // RB) + r, 0)),
        out_shape=jax.ShapeDtypeStruct((NSTEP * N, NHID), jnp.float32),
    )(x, W1)


NBUF = 2
EB = ET // 2
CHB = EB // C
NGB = CHB // NBUF


def _sc_spmm(nsteps, sup, srcp, dstp):
    mesh = plsc.VectorSubcoreMesh(core_axis_name="c", subcore_axis_name="s")

    @functools.partial(
        pl.kernel,
        mesh=mesh,
        out_type=jax.ShapeDtypeStruct((nsteps * NC * NP, NHID), jnp.float32),
        scratch_types=[
            pltpu.VMEM((EB,), jnp.int32),
            pltpu.VMEM((EB,), jnp.int32),
            pltpu.VMEM((NBUF, C, NHID), jnp.float32),
            pltpu.VMEM((ZR, NHID), jnp.float32),
            pltpu.VMEM_SHARED((NP, NHID), jnp.float32),
        ] + [pltpu.SemaphoreType.DMA] * NBUF,
    )
    def k(sup_hbm, src_hbm, dst_hbm, out_hbm, src_v, dst_v, rows_v, zbuf,
          acc, *sems):
        c = lax.axis_index("c")
        s = lax.axis_index("s")

        def issue_gather(g, b):
            pltpu.async_copy(sup_hbm.at[src_v.at[pl.ds(g * C, C)]],
                             rows_v.at[b], sems[b])

        def wait_gather(b):
            pltpu.make_async_copy(sup_hbm.at[pl.ds(0, C)], rows_v.at[b],
                                  sems[b]).wait()

        def sync_scatter(g, b):
            pltpu.sync_copy(rows_v.at[b], acc.at[dst_v.at[pl.ds(g * C, C)]],
                            add=True)

        @pl.loop(0, ZR)
        def _(i):
            for j in range(NHID // LANES):
                zbuf[i, pl.ds(j * LANES, LANES)] = jnp.zeros(
                    (LANES,), jnp.float32)

        for st in range(nsteps):
            for kk in range(RPT // ZR):
                pltpu.sync_copy(zbuf, acc.at[pl.ds(s * RPT + kk * ZR, ZR)])
            plsc.subcore_barrier()

            base = ((st * NC + c) * NS + s) * ET
            for blk in range(ET // EB):
                boff = base + blk * EB
                pltpu.sync_copy(src_hbm.at[pl.ds(boff, EB)], src_v)
                pltpu.sync_copy(dst_hbm.at[pl.ds(boff, EB)], dst_v)

                for b in range(NBUF):
                    issue_gather(b, b)

                @pl.loop(0, NGB)
                def _(gg):
                    for b in range(NBUF):
                        g = gg * NBUF + b
                        wait_gather(b)
                        sync_scatter(g, b)

                        @pl.when(g + NBUF < CHB)
                        def _():
                            issue_gather(g + NBUF, b)

            plsc.subcore_barrier()
            pltpu.sync_copy(acc.at[pl.ds(s * RPT, RPT)],
                            out_hbm.at[pl.ds((st * NC + c) * NP + s * RPT,
                                             RPT)])

    return k(sup, srcp, dstp)


def _mid_body(p_ref, b1_ref, w_ref, o_ref):
    o = jnp.zeros((RB, NHID), jnp.float32)
    for s in range(NSTEP):
        h = jnp.maximum(p_ref[s, 0] + p_ref[s, 1] + b1_ref[s][None, :], 0.0)
        o = o + jnp.dot(h, w_ref[s], preferred_element_type=jnp.float32)
    o_ref[...] = o


def _mid(agg, b1, W2p):
    return pl.pallas_call(
        _mid_body,
        grid=(N // RB,),
        in_specs=[
            pl.BlockSpec((NSTEP, NC, RB, NHID), lambda r: (0, 0, r, 0)),
            pl.BlockSpec((NSTEP, NHID), lambda r: (0, 0)),
            pl.BlockSpec((NSTEP, NHID, NHID), lambda r: (0, 0, 0)),
        ],
        out_specs=pl.BlockSpec((RB, NHID), lambda r: (r, 0)),
        out_shape=jax.ShapeDtypeStruct((N, NHID), jnp.float32),
    )(agg, b1, W2p)


def _final_body(p_ref, b2_ref, o_ref):
    a = p_ref[0, :, :NCLASS] + p_ref[1, :, :NCLASS] + b2_ref[...]
    m = jnp.max(a, axis=1, keepdims=True)
    ex = jnp.exp(a - m)
    lse = jnp.log(jnp.sum(ex, axis=1, keepdims=True))
    o_ref[...] = a - m - lse


def _final(parts, b2):
    return pl.pallas_call(
        _final_body,
        grid=(N // RB,),
        in_specs=[
            pl.BlockSpec((NC, RB, NHID), lambda r: (0, r, 0)),
            pl.BlockSpec((1, NCLASS), lambda r: (0, 0)),
        ],
        out_specs=pl.BlockSpec((RB, NCLASS), lambda r: (r, 0)),
        out_shape=jax.ShapeDtypeStruct((N, NCLASS), jnp.float32),
    )(parts, b2)


def _pad_edges(src, dst):
    srcp = jnp.pad(src, ((0, 0), (0, 0), (0, ET - ET0)))
    dstp = jnp.pad(dst, ((0, 0), (0, 0), (0, ET - ET0)), constant_values=N)
    return srcp.reshape(-1), dstp.reshape(-1)


def kernel(x, adjs, W1, b1, W2, b2):
    sup1 = _supports(x, W1)

    step_off = (jnp.arange(NSTEP, dtype=jnp.int32) * N)[:, None]
    src1 = (adjs[:, 0, :] + step_off).reshape(NSTEP, NC * NS, ET0)
    dst1 = adjs[:, 1, :].reshape(NSTEP, NC * NS, ET0)
    src1, dst1 = _pad_edges(src1, dst1)
    agg = _sc_spmm(NSTEP, sup1, src1, dst1).reshape(NSTEP, NC, NP, NHID)

    W2p = jnp.pad(W2.reshape(NSTEP, NHID, NCLASS),
                  ((0, 0), (0, 0), (0, NHID - NCLASS)))
    sup2 = _mid(agg, b1, W2p)

    src2 = adjs[0, 0].reshape(1, NC * NS, ET0)
    dst2 = adjs[0, 1].reshape(1, NC * NS, ET0)
    src2, dst2 = _pad_edges(src2, dst2)
    parts = _sc_spmm(1, sup2, src2, dst2).reshape(NC, NP, NHID)

    return _final(parts, b2.reshape(1, NCLASS))
